# baseline (speedup 1.0000x reference)
import jax
import jax.numpy as jnp
from jax.experimental import pallas as pl
from jax.experimental.pallas import tpu as pltpu

_G = 4
_SPLIT = 1


def _gcn_kernel(x_ref, adj1_ref, adj2_ref, w1_ref, w2_ref,
                fcw_ref, fcb_ref, clsw_ref, clsb_ref, out_ref,
                w1s, w2s, fcws, a1s, a2s):
    GC = x_ref.shape[0]
    C = adj1_ref.shape[0]
    G = GC // C
    Do = w2_ref.shape[1]
    f32 = jnp.float32
    cdt = jnp.bfloat16

    @pl.when(pl.program_id(1) == 0)
    def _prep():
        w1s[...] = w1_ref[...].astype(cdt)
        w2s[...] = w2_ref[...].astype(cdt)
        fcws[...] = fcw_ref[...].astype(cdt)
        a1 = adj1_ref[...].astype(cdt)
        a2 = adj2_ref[...].astype(cdt)
        a1s[0:C, :] = a1
        a1s[C:2 * C, :] = a2
        z = jnp.zeros((C, C), cdt)
        a2s[0:C, 0:C] = a1
        a2s[0:C, C:2 * C] = z
        a2s[C:2 * C, 0:C] = z
        a2s[C:2 * C, C:2 * C] = a2

    x = x_ref[...].astype(cdt)

    s1 = jnp.dot(x, w1s[...], preferred_element_type=f32)
    s1 = s1.astype(cdt)

    a1 = a1s[...]
    h1 = jnp.concatenate(
        [jnp.dot(a1, s1[g * C:(g + 1) * C], preferred_element_type=f32)
         for g in range(G)], axis=0)
    h1 = jnp.maximum(h1, 0.2 * h1)

    s2 = jnp.dot(h1.astype(cdt), w2s[...],
                 preferred_element_type=f32).astype(cdt)

    a2 = a2s[...]
    f1s, f2s = [], []
    for g in range(G):
        gg = jnp.dot(a2, s2[g * 2 * C:(g + 1) * 2 * C],
                     preferred_element_type=f32)
        f1s.append(gg[:C])
        f2s.append(gg[C:])
    f1 = jnp.concatenate(f1s, axis=0).astype(cdt)
    f2 = jnp.concatenate(f2s, axis=0).astype(cdt)

    dn = (((1,), (1,)), ((), ()))
    fcw = fcws[...]
    clsw = clsw_ref[...]
    clsb = clsb_ref[...]
    H = GC // 2
    for h in range(2):
        r0, r1 = h * H, (h + 1) * H
        pre = (jax.lax.dot_general(f1[r0:r1], fcw[:, 0:Do], dn,
                                   preferred_element_type=f32)
               + jax.lax.dot_general(f2[r0:r1], fcw[:, Do:2 * Do], dn,
                                     preferred_element_type=f32)
               + jax.lax.dot_general(x[r0:r1], fcw[:, 2 * Do:], dn,
                                     preferred_element_type=f32)
               + fcb_ref[...])
        out = jnp.tanh(pre)
        for g in range(H // C):
            blk = out[g * C:(g + 1) * C]
            out_ref[r0 + g * C:r0 + (g + 1) * C, :] = (
                jnp.sum(blk * clsw, axis=-1, keepdims=True) + clsb)


def kernel(x_input, semantic_feature, adj1, adj2, gc_w1, gc_w2,
           fc_w, fc_b, cls_w, cls_b):
    B, C, D = semantic_feature.shape
    Dm = gc_w1.shape[1]
    Do = gc_w2.shape[1]
    Dout = fc_w.shape[0]
    G = _G
    S = _SPLIT
    J = B // G // S
    f32 = jnp.float32
    cdt = jnp.bfloat16

    sem_flat = semantic_feature.reshape(B * C, D)
    fcb = fc_b.reshape(1, Dout)
    clsb = cls_b.reshape(C, 1)

    out = pl.pallas_call(
        _gcn_kernel,
        out_shape=jax.ShapeDtypeStruct((B * C, 1), f32),
        grid_spec=pltpu.PrefetchScalarGridSpec(
            num_scalar_prefetch=0,
            grid=(S, J),
            in_specs=[
                pl.BlockSpec((G * C, D), lambda i, j: (i * J + j, 0)),
                pl.BlockSpec((C, C), lambda i, j: (0, 0)),
                pl.BlockSpec((C, C), lambda i, j: (0, 0)),
                pl.BlockSpec((D, Dm), lambda i, j: (0, 0)),
                pl.BlockSpec((Dm, Do), lambda i, j: (0, 0)),
                pl.BlockSpec((Dout, 2 * Do + D), lambda i, j: (0, 0)),
                pl.BlockSpec((1, Dout), lambda i, j: (0, 0)),
                pl.BlockSpec((C, Dout), lambda i, j: (0, 0)),
                pl.BlockSpec((C, 1), lambda i, j: (0, 0)),
            ],
            out_specs=pl.BlockSpec((G * C, 1), lambda i, j: (i * J + j, 0)),
            scratch_shapes=[
                pltpu.VMEM((D, Dm), cdt),
                pltpu.VMEM((Dm, Do), cdt),
                pltpu.VMEM((Dout, 2 * Do + D), cdt),
                pltpu.VMEM((2 * C, C), cdt),
                pltpu.VMEM((2 * C, 2 * C), cdt),
            ],
        ),
        compiler_params=pltpu.CompilerParams(
            dimension_semantics=("parallel", "arbitrary"),
            vmem_limit_bytes=60 << 20,
        ),
    )(sem_flat, adj1, adj2, gc_w1, gc_w2, fc_w, fcb, cls_w, clsb)

    return out.reshape(B, C)


# speedup vs baseline: 1.0357x; 1.0357x over previous
import jax
import jax.numpy as jnp
from jax.experimental import pallas as pl
from jax.experimental.pallas import tpu as pltpu

_G = 8
_SPLIT = 1


def _gcn_kernel(x_ref, adj1_ref, adj2_ref, w1_ref, w2_ref,
                fcw_ref, fcb_ref, clsw_ref, clsb_ref, out_ref,
                w1s, w2s, fcws, a1s, a2s):
    GC = x_ref.shape[0]
    C = adj1_ref.shape[0]
    G = GC // C
    Do = w2_ref.shape[1]
    f32 = jnp.float32
    cdt = jnp.bfloat16

    @pl.when(pl.program_id(1) == 0)
    def _prep():
        w1s[...] = w1_ref[...].astype(cdt)
        w2s[...] = w2_ref[...].astype(cdt)
        fcws[...] = fcw_ref[...].astype(cdt)
        a1 = adj1_ref[...].astype(cdt)
        a2 = adj2_ref[...].astype(cdt)
        a1s[0:C, :] = a1
        a1s[C:2 * C, :] = a2
        z = jnp.zeros((C, C), cdt)
        a2s[0:C, 0:C] = a1
        a2s[0:C, C:2 * C] = z
        a2s[C:2 * C, 0:C] = z
        a2s[C:2 * C, C:2 * C] = a2

    x = x_ref[...].astype(cdt)

    s1 = jnp.dot(x, w1s[...], preferred_element_type=f32)
    s1 = s1.astype(cdt)

    a1 = a1s[...]
    h1 = jnp.concatenate(
        [jnp.dot(a1, s1[g * C:(g + 1) * C], preferred_element_type=f32)
         for g in range(G)], axis=0)
    h1 = jnp.maximum(h1, 0.2 * h1)

    s2 = jnp.dot(h1.astype(cdt), w2s[...],
                 preferred_element_type=f32).astype(cdt)

    a2 = a2s[...]
    f1s, f2s = [], []
    for g in range(G):
        gg = jnp.dot(a2, s2[g * 2 * C:(g + 1) * 2 * C],
                     preferred_element_type=f32)
        f1s.append(gg[:C])
        f2s.append(gg[C:])
    f1 = jnp.concatenate(f1s, axis=0).astype(cdt)
    f2 = jnp.concatenate(f2s, axis=0).astype(cdt)

    dn = (((1,), (1,)), ((), ()))
    fcw = fcws[...]
    clsw = clsw_ref[...]
    clsb = clsb_ref[...]
    H = GC // 4
    for h in range(4):
        r0, r1 = h * H, (h + 1) * H
        pre = (jax.lax.dot_general(f1[r0:r1], fcw[:, 0:Do], dn,
                                   preferred_element_type=f32)
               + jax.lax.dot_general(f2[r0:r1], fcw[:, Do:2 * Do], dn,
                                     preferred_element_type=f32)
               + jax.lax.dot_general(x[r0:r1], fcw[:, 2 * Do:], dn,
                                     preferred_element_type=f32)
               + fcb_ref[...])
        out = jnp.tanh(pre)
        for g in range(H // C):
            blk = out[g * C:(g + 1) * C]
            out_ref[r0 + g * C:r0 + (g + 1) * C, :] = (
                jnp.sum(blk * clsw, axis=-1, keepdims=True) + clsb)


def kernel(x_input, semantic_feature, adj1, adj2, gc_w1, gc_w2,
           fc_w, fc_b, cls_w, cls_b):
    B, C, D = semantic_feature.shape
    Dm = gc_w1.shape[1]
    Do = gc_w2.shape[1]
    Dout = fc_w.shape[0]
    G = _G
    S = _SPLIT
    J = B // G // S
    f32 = jnp.float32
    cdt = jnp.bfloat16

    sem_flat = semantic_feature.reshape(B * C, D)
    fcb = fc_b.reshape(1, Dout)
    clsb = cls_b.reshape(C, 1)

    out = pl.pallas_call(
        _gcn_kernel,
        out_shape=jax.ShapeDtypeStruct((B * C, 1), f32),
        grid_spec=pltpu.PrefetchScalarGridSpec(
            num_scalar_prefetch=0,
            grid=(S, J),
            in_specs=[
                pl.BlockSpec((G * C, D), lambda i, j: (i * J + j, 0)),
                pl.BlockSpec((C, C), lambda i, j: (0, 0)),
                pl.BlockSpec((C, C), lambda i, j: (0, 0)),
                pl.BlockSpec((D, Dm), lambda i, j: (0, 0)),
                pl.BlockSpec((Dm, Do), lambda i, j: (0, 0)),
                pl.BlockSpec((Dout, 2 * Do + D), lambda i, j: (0, 0)),
                pl.BlockSpec((1, Dout), lambda i, j: (0, 0)),
                pl.BlockSpec((C, Dout), lambda i, j: (0, 0)),
                pl.BlockSpec((C, 1), lambda i, j: (0, 0)),
            ],
            out_specs=pl.BlockSpec((G * C, 1), lambda i, j: (i * J + j, 0)),
            scratch_shapes=[
                pltpu.VMEM((D, Dm), cdt),
                pltpu.VMEM((Dm, Do), cdt),
                pltpu.VMEM((Dout, 2 * Do + D), cdt),
                pltpu.VMEM((2 * C, C), cdt),
                pltpu.VMEM((2 * C, 2 * C), cdt),
            ],
        ),
        compiler_params=pltpu.CompilerParams(
            dimension_semantics=("parallel", "arbitrary"),
            vmem_limit_bytes=60 << 20,
        ),
    )(sem_flat, adj1, adj2, gc_w1, gc_w2, fc_w, fcb, cls_w, clsb)

    return out.reshape(B, C)


# speedup vs baseline: 1.5073x; 1.4554x over previous
import jax
import jax.numpy as jnp
from jax.experimental import pallas as pl
from jax.experimental.pallas import tpu as pltpu

_G = 8
_SPLIT = 1


def _gcn_kernel(x_ref, adj1_ref, adj2_ref, w1_ref, w2_ref,
                fcw_ref, fcb_ref, clsw_ref, clsb_ref, out_ref,
                w1s, w2s, fcws, a1s, a2s):
    GC = x_ref.shape[0]
    C = adj1_ref.shape[0]
    G = GC // C
    Do = w2_ref.shape[1]
    f32 = jnp.float32
    cdt = jnp.bfloat16

    @pl.when(pl.program_id(1) == 0)
    def _prep():
        w1s[...] = w1_ref[...].astype(cdt)
        w2s[...] = w2_ref[...].astype(cdt)
        fcws[...] = fcw_ref[...].astype(cdt)
        a1 = adj1_ref[...].astype(cdt)
        a2 = adj2_ref[...].astype(cdt)
        a1s[0:C, :] = a1
        a1s[C:2 * C, :] = a2
        z = jnp.zeros((C, C), cdt)
        a2s[0:C, 0:C] = a1
        a2s[0:C, C:2 * C] = z
        a2s[C:2 * C, 0:C] = z
        a2s[C:2 * C, C:2 * C] = a2

    x = x_ref[...].astype(cdt)

    s1 = jnp.dot(x, w1s[...], preferred_element_type=f32)
    s1 = s1.astype(cdt)

    a1 = a1s[...]
    h1 = jnp.concatenate(
        [jnp.dot(a1, s1[g * C:(g + 1) * C], preferred_element_type=f32)
         for g in range(G)], axis=0)
    h1 = jnp.maximum(h1, 0.2 * h1)

    s2 = jnp.dot(h1.astype(cdt), w2s[...],
                 preferred_element_type=f32).astype(cdt)

    a2 = a2s[...]
    f1s, f2s = [], []
    for g in range(G):
        gg = jnp.dot(a2, s2[g * 2 * C:(g + 1) * 2 * C],
                     preferred_element_type=f32)
        f1s.append(gg[:C])
        f2s.append(gg[C:])
    f1 = jnp.concatenate(f1s, axis=0).astype(cdt)
    f2 = jnp.concatenate(f2s, axis=0).astype(cdt)

    dn = (((1,), (1,)), ((), ()))
    fcw = fcws[...]
    clsw = clsw_ref[...]
    clsb = clsb_ref[...]
    fcb = fcb_ref[...]
    Dout = fcw_ref.shape[0]
    Dh = Dout // 2
    partial = [None] * G
    for h in range(2):
        c0, c1 = h * Dh, (h + 1) * Dh
        pre = (jax.lax.dot_general(f1, fcw[c0:c1, 0:Do], dn,
                                   preferred_element_type=f32)
               + jax.lax.dot_general(f2, fcw[c0:c1, Do:2 * Do], dn,
                                     preferred_element_type=f32)
               + jax.lax.dot_general(x, fcw[c0:c1, 2 * Do:], dn,
                                     preferred_element_type=f32)
               + fcb[:, c0:c1])
        out = jnp.tanh(pre)
        for g in range(G):
            p = jnp.sum(out[g * C:(g + 1) * C] * clsw[:, c0:c1],
                        axis=-1, keepdims=True)
            partial[g] = p if h == 0 else partial[g] + p
    for g in range(G):
        out_ref[g * C:(g + 1) * C, :] = partial[g] + clsb


def kernel(x_input, semantic_feature, adj1, adj2, gc_w1, gc_w2,
           fc_w, fc_b, cls_w, cls_b):
    B, C, D = semantic_feature.shape
    Dm = gc_w1.shape[1]
    Do = gc_w2.shape[1]
    Dout = fc_w.shape[0]
    G = _G
    S = _SPLIT
    J = B // G // S
    f32 = jnp.float32
    cdt = jnp.bfloat16

    sem_flat = semantic_feature.reshape(B * C, D)
    fcb = fc_b.reshape(1, Dout)
    clsb = cls_b.reshape(C, 1)

    out = pl.pallas_call(
        _gcn_kernel,
        out_shape=jax.ShapeDtypeStruct((B * C, 1), f32),
        grid_spec=pltpu.PrefetchScalarGridSpec(
            num_scalar_prefetch=0,
            grid=(S, J),
            in_specs=[
                pl.BlockSpec((G * C, D), lambda i, j: (i * J + j, 0)),
                pl.BlockSpec((C, C), lambda i, j: (0, 0)),
                pl.BlockSpec((C, C), lambda i, j: (0, 0)),
                pl.BlockSpec((D, Dm), lambda i, j: (0, 0)),
                pl.BlockSpec((Dm, Do), lambda i, j: (0, 0)),
                pl.BlockSpec((Dout, 2 * Do + D), lambda i, j: (0, 0)),
                pl.BlockSpec((1, Dout), lambda i, j: (0, 0)),
                pl.BlockSpec((C, Dout), lambda i, j: (0, 0)),
                pl.BlockSpec((C, 1), lambda i, j: (0, 0)),
            ],
            out_specs=pl.BlockSpec((G * C, 1), lambda i, j: (i * J + j, 0)),
            scratch_shapes=[
                pltpu.VMEM((D, Dm), cdt),
                pltpu.VMEM((Dm, Do), cdt),
                pltpu.VMEM((Dout, 2 * Do + D), cdt),
                pltpu.VMEM((2 * C, C), cdt),
                pltpu.VMEM((2 * C, 2 * C), cdt),
            ],
        ),
        compiler_params=pltpu.CompilerParams(
            dimension_semantics=("parallel", "arbitrary"),
            vmem_limit_bytes=60 << 20,
        ),
    )(sem_flat, adj1, adj2, gc_w1, gc_w2, fc_w, fcb, cls_w, clsb)

    return out.reshape(B, C)


# speedup vs baseline: 1.5164x; 1.0061x over previous
import jax
import jax.numpy as jnp
from jax.experimental import pallas as pl
from jax.experimental.pallas import tpu as pltpu

_G = 8
_SPLIT = 1


def _gcn_kernel(x_ref, adj1_ref, adj2_ref, w1_ref, w2_ref,
                fcw_ref, fcb_ref, clsw_ref, clsb_ref, out_ref,
                w1s, w2s, fcws, a1s, a2s):
    GC = x_ref.shape[0]
    C = adj1_ref.shape[0]
    G = GC // C
    Do = w2_ref.shape[1]
    f32 = jnp.float32
    cdt = jnp.bfloat16

    @pl.when(pl.program_id(1) == 0)
    def _prep():
        w1s[...] = w1_ref[...].astype(cdt)
        w2s[...] = w2_ref[...].astype(cdt)
        fcws[...] = fcw_ref[...].astype(cdt)
        a1 = adj1_ref[...].astype(cdt)
        a2 = adj2_ref[...].astype(cdt)
        a1s[0:C, :] = a1
        a1s[C:2 * C, :] = a2
        z = jnp.zeros((C, C), cdt)
        a2s[0:C, 0:C] = a1
        a2s[0:C, C:2 * C] = z
        a2s[C:2 * C, 0:C] = z
        a2s[C:2 * C, C:2 * C] = a2

    x = x_ref[...].astype(cdt)

    s1 = jnp.dot(x, w1s[...], preferred_element_type=f32)
    s1 = s1.astype(cdt)

    a1 = a1s[...]
    h1 = jnp.concatenate(
        [jnp.dot(a1, s1[g * C:(g + 1) * C], preferred_element_type=f32)
         for g in range(G)], axis=0)
    h1 = jnp.maximum(h1, 0.2 * h1)

    s2 = jnp.dot(h1.astype(cdt), w2s[...],
                 preferred_element_type=f32).astype(cdt)

    a2 = a2s[...]
    f1s, f2s = [], []
    for g in range(G):
        gg = jnp.dot(a2, s2[g * 2 * C:(g + 1) * 2 * C],
                     preferred_element_type=f32)
        f1s.append(gg[:C])
        f2s.append(gg[C:])
    f1 = jnp.concatenate(f1s, axis=0).astype(cdt)
    f2 = jnp.concatenate(f2s, axis=0).astype(cdt)

    dn = (((1,), (1,)), ((), ()))
    fcw = fcws[...]
    clsw = clsw_ref[...]
    clsb = clsb_ref[...]
    H = GC // 2
    for h in range(2):
        r0, r1 = h * H, (h + 1) * H
        pre = (jax.lax.dot_general(f1[r0:r1], fcw[:, 0:Do], dn,
                                   preferred_element_type=f32)
               + jax.lax.dot_general(f2[r0:r1], fcw[:, Do:2 * Do], dn,
                                     preferred_element_type=f32)
               + jax.lax.dot_general(x[r0:r1], fcw[:, 2 * Do:], dn,
                                     preferred_element_type=f32)
               + fcb_ref[...])
        out = jnp.tanh(pre)
        for g in range(H // C):
            blk = out[g * C:(g + 1) * C]
            out_ref[r0 + g * C:r0 + (g + 1) * C, :] = (
                jnp.sum(blk * clsw, axis=-1, keepdims=True) + clsb)


def kernel(x_input, semantic_feature, adj1, adj2, gc_w1, gc_w2,
           fc_w, fc_b, cls_w, cls_b):
    B, C, D = semantic_feature.shape
    Dm = gc_w1.shape[1]
    Do = gc_w2.shape[1]
    Dout = fc_w.shape[0]
    G = _G
    S = _SPLIT
    J = B // G // S
    f32 = jnp.float32
    cdt = jnp.bfloat16

    sem_flat = semantic_feature.reshape(B * C, D)
    fcb = fc_b.reshape(1, Dout)
    clsb = cls_b.reshape(C, 1)

    out = pl.pallas_call(
        _gcn_kernel,
        out_shape=jax.ShapeDtypeStruct((B * C, 1), f32),
        grid_spec=pltpu.PrefetchScalarGridSpec(
            num_scalar_prefetch=0,
            grid=(S, J),
            in_specs=[
                pl.BlockSpec((G * C, D), lambda i, j: (i * J + j, 0)),
                pl.BlockSpec((C, C), lambda i, j: (0, 0)),
                pl.BlockSpec((C, C), lambda i, j: (0, 0)),
                pl.BlockSpec((D, Dm), lambda i, j: (0, 0)),
                pl.BlockSpec((Dm, Do), lambda i, j: (0, 0)),
                pl.BlockSpec((Dout, 2 * Do + D), lambda i, j: (0, 0)),
                pl.BlockSpec((1, Dout), lambda i, j: (0, 0)),
                pl.BlockSpec((C, Dout), lambda i, j: (0, 0)),
                pl.BlockSpec((C, 1), lambda i, j: (0, 0)),
            ],
            out_specs=pl.BlockSpec((G * C, 1), lambda i, j: (i * J + j, 0)),
            scratch_shapes=[
                pltpu.VMEM((D, Dm), cdt),
                pltpu.VMEM((Dm, Do), cdt),
                pltpu.VMEM((Dout, 2 * Do + D), cdt),
                pltpu.VMEM((2 * C, C), cdt),
                pltpu.VMEM((2 * C, 2 * C), cdt),
            ],
        ),
        compiler_params=pltpu.CompilerParams(
            dimension_semantics=("parallel", "arbitrary"),
            vmem_limit_bytes=60 << 20,
        ),
    )(sem_flat, adj1, adj2, gc_w1, gc_w2, fc_w, fcb, cls_w, clsb)

    return out.reshape(B, C)


# speedup vs baseline: 1.5176x; 1.0008x over previous
import jax
import jax.numpy as jnp
from jax.experimental import pallas as pl
from jax.experimental.pallas import tpu as pltpu

_G = 8
_SPLIT = 1


def _gcn_kernel(x_ref, adj1_ref, adj2_ref, w1_ref, w2_ref,
                fcw_ref, fcb_ref, clsw_ref, clsb_ref, out_ref,
                w1s, w2s, fcws, a1s, a2s):
    GC = x_ref.shape[0]
    C = adj1_ref.shape[0]
    G = GC // C
    Do = w2_ref.shape[1]
    f32 = jnp.float32
    cdt = jnp.bfloat16

    @pl.when(pl.program_id(1) == 0)
    def _prep():
        w1s[...] = w1_ref[...].astype(cdt)
        w2s[...] = w2_ref[...].astype(cdt)
        fcws[...] = fcw_ref[...].astype(cdt)
        a1 = adj1_ref[...].astype(cdt)
        a2 = adj2_ref[...].astype(cdt)
        a1s[0:C, :] = a1
        a1s[C:2 * C, :] = a2
        z = jnp.zeros((C, C), cdt)
        a2s[0:C, 0:C] = a1
        a2s[0:C, C:2 * C] = z
        a2s[C:2 * C, 0:C] = z
        a2s[C:2 * C, C:2 * C] = a2

    x = x_ref[...].astype(cdt)

    s1 = jnp.dot(x, w1s[...], preferred_element_type=f32)
    s1 = s1.astype(cdt)

    a1 = a1s[...]
    h1 = jnp.concatenate(
        [jnp.dot(a1, s1[g * C:(g + 1) * C], preferred_element_type=f32)
         for g in range(G)], axis=0)
    h1 = jnp.maximum(h1, 0.2 * h1)

    s2 = jnp.dot(h1.astype(cdt), w2s[...],
                 preferred_element_type=f32)

    a2 = a2s[...]
    f1s, f2s = [], []
    for g in range(G):
        gg = jnp.dot(a2, s2[g * 2 * C:(g + 1) * 2 * C],
                     preferred_element_type=f32)
        f1s.append(gg[:C])
        f2s.append(gg[C:])
    f1 = jnp.concatenate(f1s, axis=0).astype(cdt)
    f2 = jnp.concatenate(f2s, axis=0).astype(cdt)

    dn = (((1,), (1,)), ((), ()))
    fcw = fcws[...]
    clsw = clsw_ref[...]
    clsb = clsb_ref[...]
    H = GC // 2
    for h in range(2):
        r0, r1 = h * H, (h + 1) * H
        pre = (jax.lax.dot_general(f1[r0:r1], fcw[:, 0:Do], dn,
                                   preferred_element_type=f32)
               + jax.lax.dot_general(f2[r0:r1], fcw[:, Do:2 * Do], dn,
                                     preferred_element_type=f32)
               + jax.lax.dot_general(x[r0:r1], fcw[:, 2 * Do:], dn,
                                     preferred_element_type=f32)
               + fcb_ref[...])
        out = jnp.tanh(pre)
        for g in range(H // C):
            blk = out[g * C:(g + 1) * C]
            out_ref[r0 + g * C:r0 + (g + 1) * C, :] = (
                jnp.sum(blk * clsw, axis=-1, keepdims=True) + clsb)


def kernel(x_input, semantic_feature, adj1, adj2, gc_w1, gc_w2,
           fc_w, fc_b, cls_w, cls_b):
    B, C, D = semantic_feature.shape
    Dm = gc_w1.shape[1]
    Do = gc_w2.shape[1]
    Dout = fc_w.shape[0]
    G = _G
    S = _SPLIT
    J = B // G // S
    f32 = jnp.float32
    cdt = jnp.bfloat16

    sem_flat = semantic_feature.reshape(B * C, D)
    fcb = fc_b.reshape(1, Dout)
    clsb = cls_b.reshape(C, 1)

    out = pl.pallas_call(
        _gcn_kernel,
        out_shape=jax.ShapeDtypeStruct((B * C, 1), f32),
        grid_spec=pltpu.PrefetchScalarGridSpec(
            num_scalar_prefetch=0,
            grid=(S, J),
            in_specs=[
                pl.BlockSpec((G * C, D), lambda i, j: (i * J + j, 0)),
                pl.BlockSpec((C, C), lambda i, j: (0, 0)),
                pl.BlockSpec((C, C), lambda i, j: (0, 0)),
                pl.BlockSpec((D, Dm), lambda i, j: (0, 0)),
                pl.BlockSpec((Dm, Do), lambda i, j: (0, 0)),
                pl.BlockSpec((Dout, 2 * Do + D), lambda i, j: (0, 0)),
                pl.BlockSpec((1, Dout), lambda i, j: (0, 0)),
                pl.BlockSpec((C, Dout), lambda i, j: (0, 0)),
                pl.BlockSpec((C, 1), lambda i, j: (0, 0)),
            ],
            out_specs=pl.BlockSpec((G * C, 1), lambda i, j: (i * J + j, 0)),
            scratch_shapes=[
                pltpu.VMEM((D, Dm), cdt),
                pltpu.VMEM((Dm, Do), cdt),
                pltpu.VMEM((Dout, 2 * Do + D), cdt),
                pltpu.VMEM((2 * C, C), cdt),
                pltpu.VMEM((2 * C, 2 * C), cdt),
            ],
        ),
        compiler_params=pltpu.CompilerParams(
            dimension_semantics=("parallel", "arbitrary"),
            vmem_limit_bytes=60 << 20,
        ),
    )(sem_flat, adj1, adj2, gc_w1, gc_w2, fc_w, fcb, cls_w, clsb)

    return out.reshape(B, C)


# speedup vs baseline: 1.5214x; 1.0025x over previous
import jax
import jax.numpy as jnp
from jax.experimental import pallas as pl
from jax.experimental.pallas import tpu as pltpu

_G = 8
_SPLIT = 1


def _gcn_kernel(x_ref, adj1_ref, adj2_ref, w1_ref, w2_ref,
                fcw_ref, fcb_ref, clsw_ref, clsb_ref, out_ref,
                w1s, w2s, fcws, a1s, a2s):
    GC = x_ref.shape[0]
    C = adj1_ref.shape[0]
    G = GC // C
    Do = w2_ref.shape[1]
    f32 = jnp.float32
    cdt = jnp.bfloat16

    @pl.when(pl.program_id(1) == 0)
    def _prep():
        w1s[...] = w1_ref[...].astype(cdt)
        w2s[...] = w2_ref[...].astype(cdt)
        fcws[...] = fcw_ref[...].astype(cdt)
        a1 = adj1_ref[...].astype(cdt)
        a2 = adj2_ref[...].astype(cdt)
        a1s[0:C, :] = a1
        a1s[C:2 * C, :] = a2
        z = jnp.zeros((C, C), cdt)
        a2s[0:C, 0:C] = a1
        a2s[0:C, C:2 * C] = z
        a2s[C:2 * C, 0:C] = z
        a2s[C:2 * C, C:2 * C] = a2

    x = x_ref[...].astype(cdt)

    s1 = jnp.dot(x, w1s[...], preferred_element_type=f32)

    a1 = a1s[...]
    h1 = jnp.concatenate(
        [jnp.dot(a1, s1[g * C:(g + 1) * C], preferred_element_type=f32)
         for g in range(G)], axis=0)
    h1 = jnp.maximum(h1, 0.2 * h1)

    s2 = jnp.dot(h1.astype(cdt), w2s[...],
                 preferred_element_type=f32)

    a2 = a2s[...]
    f1s, f2s = [], []
    for g in range(G):
        gg = jnp.dot(a2, s2[g * 2 * C:(g + 1) * 2 * C],
                     preferred_element_type=f32)
        f1s.append(gg[:C])
        f2s.append(gg[C:])
    f1 = jnp.concatenate(f1s, axis=0).astype(cdt)
    f2 = jnp.concatenate(f2s, axis=0).astype(cdt)

    dn = (((1,), (1,)), ((), ()))
    fcw = fcws[...]
    clsw = clsw_ref[...]
    clsb = clsb_ref[...]
    H = GC // 2
    for h in range(2):
        r0, r1 = h * H, (h + 1) * H
        pre = (jax.lax.dot_general(f1[r0:r1], fcw[:, 0:Do], dn,
                                   preferred_element_type=f32)
               + jax.lax.dot_general(f2[r0:r1], fcw[:, Do:2 * Do], dn,
                                     preferred_element_type=f32)
               + jax.lax.dot_general(x[r0:r1], fcw[:, 2 * Do:], dn,
                                     preferred_element_type=f32)
               + fcb_ref[...])
        out = jnp.tanh(pre)
        for g in range(H // C):
            blk = out[g * C:(g + 1) * C]
            out_ref[r0 + g * C:r0 + (g + 1) * C, :] = (
                jnp.sum(blk * clsw, axis=-1, keepdims=True) + clsb)


def kernel(x_input, semantic_feature, adj1, adj2, gc_w1, gc_w2,
           fc_w, fc_b, cls_w, cls_b):
    B, C, D = semantic_feature.shape
    Dm = gc_w1.shape[1]
    Do = gc_w2.shape[1]
    Dout = fc_w.shape[0]
    G = _G
    S = _SPLIT
    J = B // G // S
    f32 = jnp.float32
    cdt = jnp.bfloat16

    sem_flat = semantic_feature.reshape(B * C, D)
    fcb = fc_b.reshape(1, Dout)
    clsb = cls_b.reshape(C, 1)

    out = pl.pallas_call(
        _gcn_kernel,
        out_shape=jax.ShapeDtypeStruct((B * C, 1), f32),
        grid_spec=pltpu.PrefetchScalarGridSpec(
            num_scalar_prefetch=0,
            grid=(S, J),
            in_specs=[
                pl.BlockSpec((G * C, D), lambda i, j: (i * J + j, 0)),
                pl.BlockSpec((C, C), lambda i, j: (0, 0)),
                pl.BlockSpec((C, C), lambda i, j: (0, 0)),
                pl.BlockSpec((D, Dm), lambda i, j: (0, 0)),
                pl.BlockSpec((Dm, Do), lambda i, j: (0, 0)),
                pl.BlockSpec((Dout, 2 * Do + D), lambda i, j: (0, 0)),
                pl.BlockSpec((1, Dout), lambda i, j: (0, 0)),
                pl.BlockSpec((C, Dout), lambda i, j: (0, 0)),
                pl.BlockSpec((C, 1), lambda i, j: (0, 0)),
            ],
            out_specs=pl.BlockSpec((G * C, 1), lambda i, j: (i * J + j, 0)),
            scratch_shapes=[
                pltpu.VMEM((D, Dm), cdt),
                pltpu.VMEM((Dm, Do), cdt),
                pltpu.VMEM((Dout, 2 * Do + D), cdt),
                pltpu.VMEM((2 * C, C), cdt),
                pltpu.VMEM((2 * C, 2 * C), cdt),
            ],
        ),
        compiler_params=pltpu.CompilerParams(
            dimension_semantics=("parallel", "arbitrary"),
            vmem_limit_bytes=60 << 20,
        ),
    )(sem_flat, adj1, adj2, gc_w1, gc_w2, fc_w, fcb, cls_w, clsb)

    return out.reshape(B, C)
